# baseline (speedup 1.0000x reference)
import jax
import jax.numpy as jnp
from jax.experimental import pallas as pl
from jax.experimental.pallas import tpu as pltpu

C = 128
H = 112
W = 152
INV_C = 1.0 / C


def _leaky(v):
    return jnp.where(v >= 0, v, 0.01 * v)


def _gathered(w_ref, oh, n_in, n_out, chunk_in):
    rows = n_in * n_out
    crows = chunk_in * n_out
    chunks = []
    for k in range(rows // crows):
        gk = jax.lax.dot_general(
            w_ref[0, :, k * crows:(k + 1) * crows], oh,
            ((( 0,), (0,)), ((), ())),
            preferred_element_type=jnp.float32)
        chunks.append(gk.reshape(chunk_in, n_out, W))
    return chunks


def _row_kernel(x_ref, xg_ref, wc_ref, bc_ref, w1_ref, w2_ref, wrc_ref,
                xo_ref, mask_ref, loss_ref):
    Bn = x_ref.shape[1]
    idxf = []
    regs = []
    for b in range(Bn):
        X = x_ref[0, b]
        xg = xg_ref[0, b]
        idx = jnp.clip((xg * C).astype(jnp.int32), 0, C - 1)
        ci = jax.lax.broadcasted_iota(jnp.int32, (C, W), 0)
        oh = (ci == idx).astype(jnp.float32)

        cls = jnp.dot(wc_ref[0], X, preferred_element_type=jnp.float32)
        cls = _leaky(cls + bc_ref[0])
        mx = jnp.max(cls, axis=0, keepdims=True)
        e = jnp.exp(cls - mx)
        p = e / jnp.sum(e, axis=0, keepdims=True)
        lse = jnp.log(jnp.sum(jnp.exp(p), axis=0, keepdims=True))
        p_gt = jnp.sum(p * oh, axis=0, keepdims=True)
        loss_ref[0, b] = (lse - p_gt)[0]

        y1 = jnp.zeros((32, W), jnp.float32)
        for k, g in enumerate(_gathered(w1_ref, oh, 128, 32, 16)):
            y1 = y1 + jnp.sum(g * X[k * 16:(k + 1) * 16, None, :], axis=0)
        y1 = _leaky(y1)

        y2 = jnp.zeros((32, W), jnp.float32)
        for k, g in enumerate(_gathered(w2_ref, oh, 32, 32, 16)):
            y2 = y2 + jnp.sum(g * y1[k * 16:(k + 1) * 16, None, :], axis=0)
        y2 = _leaky(y2)

        g3 = _gathered(wrc_ref, oh, 32, 2, 32)[0]
        y3 = jnp.sum(g3 * y2[:, None, :], axis=0)
        reg = _leaky(y3[0])
        mask_ref[0, b] = _leaky(y3[1])
        idxf.append(idx[0].astype(jnp.float32))
        regs.append(reg)

    for i in range(Bn):
        for j in range(Bn):
            xo_ref[0, i, j, :] = idxf[i] * INV_C + regs[j] * INV_C


def kernel(x, x_gt, conv_c_w, conv_c_b, w1, w2, wrc):
    B = x.shape[0]
    xr = jnp.transpose(x, (2, 0, 1, 3))
    xgr = jnp.transpose(x_gt, (2, 0, 1, 3))
    wc = conv_c_w.reshape(H, C, 128)
    bc = conv_c_b.reshape(H, C)[:, :, None]
    w1f = w1.reshape(H, C, 128 * 32)
    w2f = w2.reshape(H, C, 32 * 32)
    wrcf = wrc.reshape(H, C, 32 * 2)

    xo_t, mask_t, loss_t = pl.pallas_call(
        _row_kernel,
        grid=(H,),
        in_specs=[
            pl.BlockSpec((1, B, 128, W), lambda h: (h, 0, 0, 0)),
            pl.BlockSpec((1, B, 1, W), lambda h: (h, 0, 0, 0)),
            pl.BlockSpec((1, C, 128), lambda h: (h, 0, 0)),
            pl.BlockSpec((1, C, 1), lambda h: (h, 0, 0)),
            pl.BlockSpec((1, C, 128 * 32), lambda h: (h, 0, 0)),
            pl.BlockSpec((1, C, 32 * 32), lambda h: (h, 0, 0)),
            pl.BlockSpec((1, C, 32 * 2), lambda h: (h, 0, 0)),
        ],
        out_specs=[
            pl.BlockSpec((1, B, B, W), lambda h: (h, 0, 0, 0)),
            pl.BlockSpec((1, B, W), lambda h: (h, 0, 0)),
            pl.BlockSpec((1, B, W), lambda h: (h, 0, 0)),
        ],
        out_shape=[
            jax.ShapeDtypeStruct((H, B, B, W), jnp.float32),
            jax.ShapeDtypeStruct((H, B, W), jnp.float32),
            jax.ShapeDtypeStruct((H, B, W), jnp.float32),
        ],
        compiler_params=pltpu.CompilerParams(
            dimension_semantics=("parallel",)),
    )(xr, xgr, wc, bc, w1f, w2f, wrcf)

    x_out = jnp.transpose(xo_t, (1, 2, 0, 3))
    mask = jnp.transpose(mask_t, (1, 0, 2))
    loss = jnp.transpose(loss_t, (1, 0, 2))
    return (x_out, mask, loss)


# speedup vs baseline: 1.8543x; 1.8543x over previous
import jax
import jax.numpy as jnp
from jax.experimental import pallas as pl
from jax.experimental.pallas import tpu as pltpu

C = 128
H = 112
W = 152
INV_C = 1.0 / C
_NCH = 8
_CC = C // _NCH
_RR = _CC * 32


def _leaky(v):
    return jnp.where(v >= 0, v, 0.01 * v)


def _row_kernel(x_ref, xg_ref, wc_ref, bc_ref, w1_ref, w2_ref, wrc_ref,
                xo_ref, mask_ref, loss_ref):
    Bn = x_ref.shape[1]
    idxf = []
    regs = []
    for b in range(Bn):
        X = x_ref[0, b]
        xg = xg_ref[0, b]
        idx = jnp.clip((xg * C).astype(jnp.int32), 0, C - 1)
        ci = jax.lax.broadcasted_iota(jnp.int32, (C, W), 0)
        oh = (ci == idx).astype(jnp.float32)

        cls = jnp.dot(wc_ref[0], X, preferred_element_type=jnp.float32)
        cls = _leaky(cls + bc_ref[0])
        mx = jnp.max(cls, axis=0, keepdims=True)
        e = jnp.exp(cls - mx)
        p = e / jnp.sum(e, axis=0, keepdims=True)
        lse = jnp.log(jnp.sum(jnp.exp(p), axis=0, keepdims=True))
        p_gt = jnp.sum(p * oh, axis=0, keepdims=True)
        loss_ref[0, b] = (lse - p_gt)[0]

        y1 = jnp.zeros((32, W), jnp.float32)
        for k in range(_NCH):
            mk = jnp.dot(w1_ref[0, k * _RR:(k + 1) * _RR, :], X,
                         preferred_element_type=jnp.float32)
            mk3 = mk.reshape(_CC, 32, W)
            ohk = oh[k * _CC:(k + 1) * _CC, :]
            y1 = y1 + jnp.sum(mk3 * ohk[:, None, :], axis=0)
        y1 = _leaky(y1)

        y2 = jnp.zeros((32, W), jnp.float32)
        for k in range(_NCH):
            mk = jnp.dot(w2_ref[0, k * _RR:(k + 1) * _RR, :], y1,
                         preferred_element_type=jnp.float32)
            mk3 = mk.reshape(_CC, 32, W)
            ohk = oh[k * _CC:(k + 1) * _CC, :]
            y2 = y2 + jnp.sum(mk3 * ohk[:, None, :], axis=0)
        y2 = _leaky(y2)

        m3 = jnp.dot(wrc_ref[0], y2, preferred_element_type=jnp.float32)
        ri = jax.lax.broadcasted_iota(jnp.int32, (2 * C, W), 0)
        selc = ri // 2 == idx
        reg = jnp.sum(jnp.where(selc & (ri % 2 == 0), m3, 0.0), axis=0)
        msk = jnp.sum(jnp.where(selc & (ri % 2 == 1), m3, 0.0), axis=0)
        mask_ref[0, b] = _leaky(msk)
        idxf.append(idx[0].astype(jnp.float32))
        regs.append(_leaky(reg))

    for i in range(Bn):
        for j in range(Bn):
            xo_ref[0, i, j, :] = idxf[i] * INV_C + regs[j] * INV_C


def kernel(x, x_gt, conv_c_w, conv_c_b, w1, w2, wrc):
    B = x.shape[0]
    xr = jnp.transpose(x, (2, 0, 1, 3))
    xgr = jnp.transpose(x_gt, (2, 0, 1, 3))
    wc = conv_c_w.reshape(H, C, 128)
    bc = conv_c_b.reshape(H, C)[:, :, None]
    w1n = w1.reshape(H, C, 128, 32).transpose(0, 1, 3, 2).reshape(H, C * 32, 128)
    w2n = w2.reshape(H, C, 32, 32).transpose(0, 1, 3, 2).reshape(H, C * 32, 32)
    wrcn = wrc.reshape(H, C, 32, 2).transpose(0, 1, 3, 2).reshape(H, C * 2, 32)

    xo_t, mask_t, loss_t = pl.pallas_call(
        _row_kernel,
        grid=(H,),
        in_specs=[
            pl.BlockSpec((1, B, 128, W), lambda h: (h, 0, 0, 0)),
            pl.BlockSpec((1, B, 1, W), lambda h: (h, 0, 0, 0)),
            pl.BlockSpec((1, C, 128), lambda h: (h, 0, 0)),
            pl.BlockSpec((1, C, 1), lambda h: (h, 0, 0)),
            pl.BlockSpec((1, C * 32, 128), lambda h: (h, 0, 0)),
            pl.BlockSpec((1, C * 32, 32), lambda h: (h, 0, 0)),
            pl.BlockSpec((1, C * 2, 32), lambda h: (h, 0, 0)),
        ],
        out_specs=[
            pl.BlockSpec((1, B, B, W), lambda h: (h, 0, 0, 0)),
            pl.BlockSpec((1, B, W), lambda h: (h, 0, 0)),
            pl.BlockSpec((1, B, W), lambda h: (h, 0, 0)),
        ],
        out_shape=[
            jax.ShapeDtypeStruct((H, B, B, W), jnp.float32),
            jax.ShapeDtypeStruct((H, B, W), jnp.float32),
            jax.ShapeDtypeStruct((H, B, W), jnp.float32),
        ],
        compiler_params=pltpu.CompilerParams(
            dimension_semantics=("parallel",)),
    )(xr, xgr, wc, bc, w1n, w2n, wrcn)

    x_out = jnp.transpose(xo_t, (1, 2, 0, 3))
    mask = jnp.transpose(mask_t, (1, 0, 2))
    loss = jnp.transpose(loss_t, (1, 0, 2))
    return (x_out, mask, loss)
